# baseline (speedup 1.0000x reference)
import jax
import jax.numpy as jnp
from jax import lax
from jax.experimental import pallas as pl

S, D, E = 4096, 2048, 64
CAP = 64
SHIFT = 2048
GA, BA = 4, 1024
GB, BB = 32, 128
GD, BD = 16, 256


def _gate_body(x_ref, w_ref, g_ref, e_ref, st_ref):
    i = pl.program_id(0)
    l = jnp.dot(x_ref[...], w_ref[...], preferred_element_type=jnp.float32)
    m = jnp.max(l, axis=1, keepdims=True)
    ee = jnp.exp(l - m)
    s = jnp.sum(ee, axis=1, keepdims=True)
    gates = ee / s
    gm = jnp.max(gates, axis=1, keepdims=True)
    idx = lax.broadcasted_iota(jnp.int32, gates.shape, 1)
    eid = jnp.min(jnp.where(gates == gm, idx, E), axis=1, keepdims=True)
    g_ref[...] = gm
    e_ref[...] = eid
    sel = (idx == eid).astype(jnp.float32)
    mep = jnp.sum(gates, axis=0, keepdims=True)
    cep = jnp.sum(sel, axis=0, keepdims=True)
    pad = jnp.zeros((1, 128 - E), jnp.float32)
    block = jnp.concatenate([
        jnp.concatenate([mep, pad], axis=1),
        jnp.concatenate([cep, pad], axis=1),
        jnp.zeros((6, 128), jnp.float32),
    ], axis=0)

    @pl.when(i == 0)
    def _():
        st_ref[...] = jnp.zeros((8, 128), jnp.float32)

    st_ref[...] += block


def _rank_body(gcol_ref, gall_ref, st_ref, p_ref, laux_ref):
    i = pl.program_id(0)
    g = gcol_ref[...]
    tid = lax.broadcasted_iota(jnp.int32, (BB, 1), 0) + i * BB
    cnt = jnp.zeros((BB, 1), jnp.int32)
    for r in range(GB):
        grow = gall_ref[r:r + 1, :]
        oid = lax.broadcasted_iota(jnp.int32, (1, BB), 1) + r * BB
        hit = (grow > g) | ((grow == g) & (oid < tid))
        cnt = cnt + jnp.sum(hit.astype(jnp.int32), axis=1, keepdims=True)
    p_ref[...] = (cnt + SHIFT) & (S - 1)

    @pl.when(i == 0)
    def _():
        st = st_ref[...]
        prod = st[0:1, :] * st[1:2, :]
        laux = jnp.sum(prod) * (E * 0.01 / (float(S) * float(S)))
        laux_ref[...] = jnp.full((8, 128), laux, jnp.float32)


def _locexp_body(pcol_ref, ecol_ref, pall_ref, eall_ref, gcol_ref, cw_ref, dm_ref):
    p = pcol_ref[...]
    e = ecol_ref[...]
    cnt = jnp.zeros((BB, 1), jnp.int32)
    for r in range(GB):
        prow = pall_ref[r:r + 1, :]
        erow = eall_ref[r:r + 1, :]
        hit = (erow == e) & (prow < p)
        cnt = cnt + jnp.sum(hit.astype(jnp.int32), axis=1, keepdims=True)
    tgt = jnp.where(cnt < CAP, e * CAP + cnt + 1, 0)
    t3 = lax.broadcast_in_dim(tgt, (BB, E, CAP), (0, 1))
    g3 = lax.broadcast_in_dim(gcol_ref[...], (BB, E, CAP), (0, 1))
    f = (lax.broadcasted_iota(jnp.int32, (BB, E, CAP), 1) * CAP
         + lax.broadcasted_iota(jnp.int32, (BB, E, CAP), 2) + 1)
    hit3 = f == t3
    cw_ref[...] = jnp.where(hit3, g3, 0.0)
    dm_ref[...] = hit3


def kernel(input, wg):
    gm, eid, stats = pl.pallas_call(
        _gate_body,
        grid=(GA,),
        in_specs=[pl.BlockSpec((BA, D), lambda i: (i, 0)),
                  pl.BlockSpec((D, E), lambda i: (0, 0))],
        out_specs=(pl.BlockSpec((BA, 1), lambda i: (i, 0)),
                   pl.BlockSpec((BA, 1), lambda i: (i, 0)),
                   pl.BlockSpec((8, 128), lambda i: (0, 0))),
        out_shape=(jax.ShapeDtypeStruct((S, 1), jnp.float32),
                   jax.ShapeDtypeStruct((S, 1), jnp.int32),
                   jax.ShapeDtypeStruct((8, 128), jnp.float32)),
    )(input, wg)

    gall = gm.reshape(GB, BB)
    p, laux = pl.pallas_call(
        _rank_body,
        grid=(GB,),
        in_specs=[pl.BlockSpec((BB, 1), lambda i: (i, 0)),
                  pl.BlockSpec((GB, BB), lambda i: (0, 0)),
                  pl.BlockSpec((8, 128), lambda i: (0, 0))],
        out_specs=(pl.BlockSpec((BB, 1), lambda i: (i, 0)),
                   pl.BlockSpec((8, 128), lambda i: (0, 0))),
        out_shape=(jax.ShapeDtypeStruct((S, 1), jnp.int32),
                   jax.ShapeDtypeStruct((8, 128), jnp.float32)),
    )(gm, gall, stats)

    cw, dm = pl.pallas_call(
        _locexp_body,
        grid=(GB,),
        in_specs=[pl.BlockSpec((BB, 1), lambda i: (i, 0)),
                  pl.BlockSpec((BB, 1), lambda i: (i, 0)),
                  pl.BlockSpec((GB, BB), lambda i: (0, 0)),
                  pl.BlockSpec((GB, BB), lambda i: (0, 0)),
                  pl.BlockSpec((BB, 1), lambda i: (i, 0))],
        out_specs=(pl.BlockSpec((BB, E, CAP), lambda i: (i, 0, 0)),
                   pl.BlockSpec((BB, E, CAP), lambda i: (i, 0, 0))),
        out_shape=(jax.ShapeDtypeStruct((S, E, CAP), jnp.float32),
                   jax.ShapeDtypeStruct((S, E, CAP), jnp.bool_)),
    )(p, eid, p.reshape(GB, BB), eid.reshape(GB, BB), gm)

    return (laux[0, 0].reshape(()), cw, dm)


# speedup vs baseline: 1.7815x; 1.7815x over previous
import jax
import jax.numpy as jnp
from jax import lax
from jax.experimental import pallas as pl

S, D, E = 4096, 2048, 64
CAP = 64
SHIFT = 2048
GA, BA = 4, 1024
GB, BB = 32, 128
GD, BD = 16, 256


def _gate_body(x_ref, w_ref, g_ref, e_ref, st_ref):
    i = pl.program_id(0)
    l = jnp.dot(x_ref[...], w_ref[...], preferred_element_type=jnp.float32)
    m = jnp.max(l, axis=1, keepdims=True)
    ee = jnp.exp(l - m)
    s = jnp.sum(ee, axis=1, keepdims=True)
    gates = ee / s
    gm = jnp.max(gates, axis=1, keepdims=True)
    idx = lax.broadcasted_iota(jnp.int32, gates.shape, 1)
    eid = jnp.min(jnp.where(gates == gm, idx, E), axis=1, keepdims=True)
    g_ref[...] = gm
    e_ref[...] = eid
    sel = (idx == eid).astype(jnp.float32)
    mep = jnp.sum(gates, axis=0, keepdims=True)
    cep = jnp.sum(sel, axis=0, keepdims=True)
    pad = jnp.zeros((1, 128 - E), jnp.float32)
    block = jnp.concatenate([
        jnp.concatenate([mep, pad], axis=1),
        jnp.concatenate([cep, pad], axis=1),
        jnp.zeros((6, 128), jnp.float32),
    ], axis=0)

    @pl.when(i == 0)
    def _():
        st_ref[...] = jnp.zeros((8, 128), jnp.float32)

    st_ref[...] += block


def _rank_body(gcol_ref, gall_ref, st_ref, p_ref, laux_ref):
    i = pl.program_id(0)
    g = gcol_ref[...]
    tid = lax.broadcasted_iota(jnp.int32, (BB, 1), 0) + i * BB
    cnt = jnp.zeros((BB, 1), jnp.int32)
    for r in range(GB):
        grow = gall_ref[r:r + 1, :]
        oid = lax.broadcasted_iota(jnp.int32, (1, BB), 1) + r * BB
        hit = (grow > g) | ((grow == g) & (oid < tid))
        cnt = cnt + jnp.sum(hit.astype(jnp.int32), axis=1, keepdims=True)
    p_ref[...] = (cnt + SHIFT) & (S - 1)

    @pl.when(i == 0)
    def _():
        st = st_ref[...]
        prod = st[0:1, :] * st[1:2, :]
        laux = jnp.sum(prod) * (E * 0.01 / (float(S) * float(S)))
        laux_ref[...] = jnp.full((8, 128), laux, jnp.float32)


def _loc_body(pcol_ref, ecol_ref, pall_ref, eall_ref, t_ref):
    p = pcol_ref[...]
    e = ecol_ref[...]
    cnt = jnp.zeros((BB, 1), jnp.int32)
    for r in range(GB):
        prow = pall_ref[r:r + 1, :]
        erow = eall_ref[r:r + 1, :]
        hit = (erow == e) & (prow < p)
        cnt = cnt + jnp.sum(hit.astype(jnp.int32), axis=1, keepdims=True)
    t_ref[...] = jnp.where(cnt < CAP, e * CAP + cnt + 1, 0)


def _expand_body(t_ref, g_ref, cw_ref, dm_ref):
    t = lax.broadcast_in_dim(t_ref[...], (E, CAP, BD), (0, 1, 2))
    g = lax.broadcast_in_dim(g_ref[...], (E, CAP, BD), (0, 1, 2))
    f = (lax.broadcasted_iota(jnp.int32, (E, CAP, BD), 0) * CAP
         + lax.broadcasted_iota(jnp.int32, (E, CAP, BD), 1) + 1)
    hit = f == t
    cw_ref[...] = jnp.where(hit, g, 0.0)
    dm_ref[...] = hit


def kernel(input, wg):
    gm, eid, stats = pl.pallas_call(
        _gate_body,
        grid=(GA,),
        in_specs=[pl.BlockSpec((BA, D), lambda i: (i, 0)),
                  pl.BlockSpec((D, E), lambda i: (0, 0))],
        out_specs=(pl.BlockSpec((BA, 1), lambda i: (i, 0)),
                   pl.BlockSpec((BA, 1), lambda i: (i, 0)),
                   pl.BlockSpec((8, 128), lambda i: (0, 0))),
        out_shape=(jax.ShapeDtypeStruct((S, 1), jnp.float32),
                   jax.ShapeDtypeStruct((S, 1), jnp.int32),
                   jax.ShapeDtypeStruct((8, 128), jnp.float32)),
    )(input, wg)

    gall = gm.reshape(GB, BB)
    p, laux = pl.pallas_call(
        _rank_body,
        grid=(GB,),
        in_specs=[pl.BlockSpec((BB, 1), lambda i: (i, 0)),
                  pl.BlockSpec((GB, BB), lambda i: (0, 0)),
                  pl.BlockSpec((8, 128), lambda i: (0, 0))],
        out_specs=(pl.BlockSpec((BB, 1), lambda i: (i, 0)),
                   pl.BlockSpec((8, 128), lambda i: (0, 0))),
        out_shape=(jax.ShapeDtypeStruct((S, 1), jnp.int32),
                   jax.ShapeDtypeStruct((8, 128), jnp.float32)),
    )(gm, gall, stats)

    tgt = pl.pallas_call(
        _loc_body,
        grid=(GB,),
        in_specs=[pl.BlockSpec((BB, 1), lambda i: (i, 0)),
                  pl.BlockSpec((BB, 1), lambda i: (i, 0)),
                  pl.BlockSpec((GB, BB), lambda i: (0, 0)),
                  pl.BlockSpec((GB, BB), lambda i: (0, 0))],
        out_specs=pl.BlockSpec((BB, 1), lambda i: (i, 0)),
        out_shape=jax.ShapeDtypeStruct((S, 1), jnp.int32),
    )(p, eid, p.reshape(GB, BB), eid.reshape(GB, BB))

    t8 = tgt.reshape(GD, 1, BD)
    g8 = gm.reshape(GD, 1, BD)
    cwp, dmp = pl.pallas_call(
        _expand_body,
        grid=(GD,),
        in_specs=[pl.BlockSpec((1, 1, BD), lambda i: (i, 0, 0)),
                  pl.BlockSpec((1, 1, BD), lambda i: (i, 0, 0))],
        out_specs=(pl.BlockSpec((E, CAP, BD), lambda i: (0, 0, i)),
                   pl.BlockSpec((E, CAP, BD), lambda i: (0, 0, i))),
        out_shape=(jax.ShapeDtypeStruct((E, CAP, S), jnp.float32),
                   jax.ShapeDtypeStruct((E, CAP, S), jnp.bool_)),
    )(t8, g8)

    return (laux[0, 0].reshape(()),
            jnp.transpose(cwp, (2, 0, 1)),
            jnp.transpose(dmp, (2, 0, 1)))
